# baseline (speedup 1.0000x reference)
import functools

import jax
import jax.numpy as jnp
from jax import lax
from jax.experimental import pallas as pl
from jax.experimental.pallas import tpu as pltpu
from jax.experimental.pallas import tpu_sc as plsc

N_NODES = 10000
N_EDGES = 160000
N_GRAPHS = 1024
NE = 8
D_IN = 4
D_HID = 512
D_OUT = 128
D_GATE = 128

NC, NS, L = 2, 16, 16
NW = NC * NS
N_PAD = 10240
ROWS_PER_W = N_PAD // NW
EDGES_PER_W = N_EDGES // NW
G_PER_W = N_GRAPHS // NW
BW_PAD = 1032
LOG2N = 14

_MESH = plsc.VectorSubcoreMesh(core_axis_name="c", subcore_axis_name="s",
                               num_cores=NC, num_subcores=NS)
_SC_PARAMS = pltpu.CompilerParams(needs_layout_passes=False)

_MM_DTYPE = jnp.float32


def _wid():
    return lax.axis_index("s") * NC + lax.axis_index("c")


@functools.partial(
    pl.kernel, mesh=_MESH, compiler_params=_SC_PARAMS,
    out_type=jax.ShapeDtypeStruct((NW, D_IN * N_PAD), jnp.float32),
    scratch_types=[
        pltpu.VMEM((D_IN * N_PAD,), jnp.float32),
        pltpu.VMEM((D_IN * N_PAD,), jnp.float32),
        pltpu.VMEM((EDGES_PER_W + 8,), jnp.int32),
        pltpu.VMEM((EDGES_PER_W + 8,), jnp.int32),
        pltpu.SemaphoreType.DMA,
        pltpu.SemaphoreType.DMA,
        pltpu.SemaphoreType.DMA,
    ],
)
def _edge_scatter(x_hbm, ei_hbm, aggp_hbm, xv, av, sv, dv, smx, sms, smd):
    w = _wid()
    dx = pltpu.make_async_copy(x_hbm, xv, smx)
    dx.start()
    ds = pltpu.make_async_copy(
        ei_hbm.at[pl.ds(w * EDGES_PER_W, EDGES_PER_W)],
        sv.at[pl.ds(0, EDGES_PER_W)], sms)
    ds.start()
    dd = pltpu.make_async_copy(
        ei_hbm.at[pl.ds(N_EDGES + w * EDGES_PER_W, EDGES_PER_W)],
        dv.at[pl.ds(0, EDGES_PER_W)], smd)
    dd.start()

    def zero(i, _):
        av[pl.ds(i * L, L)] = jnp.zeros((L,), jnp.float32)
        return 0
    lax.fori_loop(0, D_IN * N_PAD // L, zero, 0)
    dx.wait()
    ds.wait()
    dd.wait()

    def group(g, mask):
        s16 = sv[pl.ds(g * L, L)]
        d16 = dv[pl.ds(g * L, L)]
        for c in range(D_IN):
            vals = plsc.load_gather(xv, [s16 + c * N_PAD], mask=mask)
            plsc.addupdate_scatter(av, [d16 + c * N_PAD], vals, mask=mask)

    def body(g, _):
        group(g, None)
        return 0
    lax.fori_loop(0, EDGES_PER_W // L, body, 0)
    group(EDGES_PER_W // L, lax.iota(jnp.int32, L) < EDGES_PER_W % L)
    pltpu.sync_copy(av, aggp_hbm.at[w])


def _gate_mlp_body(aggp_ref, xcm_ref, wg1_ref, bg1_ref, o_ref):
    agg_cm = xcm_ref[:]
    for i in range(NW):
        agg_cm = agg_cm + aggp_ref[i]
    hg = lax.dot_general(agg_cm, wg1_ref[:], (((0,), (0,)), ((), ())),
                         preferred_element_type=jnp.float32)
    o_ref[:] = jnp.maximum(hg + bg1_ref[:], 0.0)


def _gate_mlp(aggp, x_cm, Wg1, bg1):
    return pl.pallas_call(
        _gate_mlp_body,
        out_shape=jax.ShapeDtypeStruct((N_PAD, D_GATE), jnp.float32),
    )(aggp, x_cm, Wg1, bg1.reshape(1, D_GATE))


def _lower_bound_vec(bv, targets):
    def step(_, lohi):
        lo, hi = lohi
        mid = (lo + hi) // 2
        vals = plsc.load_gather(bv, [mid])
        pred = vals < targets
        return (jnp.where(pred, mid + 1, lo), jnp.where(pred, hi, mid))
    zeros = jnp.zeros((L,), jnp.int32)
    lo, _ = lax.fori_loop(0, LOG2N, step, (zeros, zeros + N_PAD))
    return lo


def _lower_bound_scalar(bv, target):
    return _lower_bound_vec(bv, jnp.full((L,), target, jnp.int32))[0]


def _make_segsum(with_cnt):
    D = D_GATE
    out_ty = [jax.ShapeDtypeStruct((N_GRAPHS, D), jnp.float32)]
    if with_cnt:
        out_ty.append(jax.ShapeDtypeStruct((N_GRAPHS,), jnp.float32))
    scratch = [
        pltpu.VMEM((N_PAD,), jnp.int32),
        pltpu.VMEM((G_PER_W, D), jnp.float32),
        pltpu.VMEM((L, D), jnp.float32),
        pltpu.VMEM((L, D), jnp.float32),
        pltpu.SemaphoreType.DMA,
        pltpu.SemaphoreType.DMA,
    ]
    if with_cnt:
        scratch.append(pltpu.VMEM((G_PER_W,), jnp.float32))

    def body(rows_hbm, batch_hbm, *refs):
        if with_cnt:
            gp_hbm, cnt_hbm, bv, acc, rb0, rb1, sem0, sem1, cntv = refs
        else:
            gp_hbm, bv, acc, rb0, rb1, sem0, sem1 = refs
        w = _wid()
        g0 = w * G_PER_W
        pltpu.sync_copy(batch_hbm, bv)
        lo = _lower_bound_scalar(bv, g0)
        hi = _lower_bound_scalar(bv, g0 + G_PER_W)
        lo8 = pl.multiple_of((lo // 8) * 8, 8)
        nch = (hi - lo8 + L - 1) // L

        def chunk_base(cidx):
            r0 = jnp.minimum(lo8 + cidx * L, N_PAD - L)
            return pl.multiple_of(r0, 8)

        def start(cidx, rb, sem):
            pltpu.make_async_copy(
                rows_hbm.at[pl.ds(chunk_base(cidx), L)], rb, sem).start()

        def process(cidx, rb):
            r0 = chunk_base(cidx)
            bchunk = bv[pl.ds(r0, L)]
            for j in range(L):
                r = r0 + j
                tgt = bchunk[j] - g0

                @pl.when((r >= lo) & (r < hi))
                def _():
                    for v in range(D // L):
                        plsc.addupdate(acc.at[tgt, pl.ds(v * L, L)],
                                       rb[j, pl.ds(v * L, L)])

        @pl.when(nch > 0)
        def _():
            start(0, rb0, sem0)

        for gl in range(G_PER_W):
            for v in range(D // L):
                acc[gl, pl.ds(v * L, L)] = jnp.zeros((L,), jnp.float32)

        def pair(m, _):
            k = m * 2

            @pl.when(k + 1 < nch)
            def _():
                start(k + 1, rb1, sem1)

            @pl.when(k < nch)
            def _():
                pltpu.make_async_copy(
                    rows_hbm.at[pl.ds(0, L)], rb0, sem0).wait()
                process(k, rb0)

            @pl.when(k + 2 < nch)
            def _():
                start(k + 2, rb0, sem0)

            @pl.when(k + 1 < nch)
            def _():
                pltpu.make_async_copy(
                    rows_hbm.at[pl.ds(0, L)], rb1, sem1).wait()
                process(k + 1, rb1)
            return 0
        lax.fori_loop(0, (nch + 1) // 2, pair, 0)
        pltpu.sync_copy(acc, gp_hbm.at[pl.ds(g0, G_PER_W)])

        if with_cnt:
            iota = lax.iota(jnp.int32, L)
            for h in range(G_PER_W // L):
                t0 = g0 + h * L + iota
                b_lo = _lower_bound_vec(bv, t0)
                b_hi = _lower_bound_vec(bv, t0 + 1)
                cntv[pl.ds(h * L, L)] = (b_hi - b_lo).astype(jnp.float32)
            pltpu.sync_copy(cntv, cnt_hbm.at[pl.ds(g0, G_PER_W)])

    return pl.kernel(body, mesh=_MESH, compiler_params=_SC_PARAMS,
                     out_type=tuple(out_ty), scratch_types=scratch)


_segsum_cnt = _make_segsum(True)
_segsum_plain = _make_segsum(False)


def _gate_head_body(gp_ref, cnt_ref, wg2_ref, bg2_ref, o_ref):
    cnt = jnp.maximum(cnt_ref[:], 1.0)
    gp = gp_ref[:] / cnt
    logits = jnp.dot(gp, wg2_ref[:], preferred_element_type=jnp.float32)
    logits = logits + bg2_ref[:]
    m = jnp.max(logits, axis=1, keepdims=True)
    ex = jnp.exp(logits - m)
    wts = ex / jnp.sum(ex, axis=1, keepdims=True)
    o_ref[:N_GRAPHS] = wts
    o_ref[N_GRAPHS:] = jnp.zeros((BW_PAD - N_GRAPHS, NE), jnp.float32)


def _gate_head(gp, cnt, Wg2, bg2):
    return pl.pallas_call(
        _gate_head_body,
        out_shape=jax.ShapeDtypeStruct((BW_PAD, NE), jnp.float32),
    )(gp, cnt.reshape(N_GRAPHS, 1), Wg2, bg2.reshape(1, NE))


@functools.partial(
    pl.kernel, mesh=_MESH, compiler_params=_SC_PARAMS,
    out_type=jax.ShapeDtypeStruct((N_PAD * NE,), jnp.float32),
    scratch_types=[
        pltpu.VMEM((BW_PAD * NE,), jnp.float32),
        pltpu.VMEM((ROWS_PER_W,), jnp.int32),
        pltpu.VMEM((ROWS_PER_W * NE,), jnp.float32),
    ],
)
def _wgather(wt_hbm, batch_hbm, out_hbm, wt, bvt, ov):
    w = _wid()
    pltpu.sync_copy(wt_hbm, wt)
    pltpu.sync_copy(batch_hbm.at[pl.ds(w * ROWS_PER_W, ROWS_PER_W)], bvt)
    iota = lax.iota(jnp.int32, L)

    def body(k, _):
        idx = bvt[pl.ds(k * L, L)] * NE
        for c in range(NE):
            vals = plsc.load_gather(wt, [idx + c])
            plsc.store_scatter(ov, [iota * NE + (k * L * NE + c)], vals)
        return 0
    lax.fori_loop(0, ROWS_PER_W // L, body, 0)
    pltpu.sync_copy(ov, out_hbm.at[pl.ds(w * ROWS_PER_W * NE, ROWS_PER_W * NE)])


_T_F = 1024
_PIECES = (4, 3, 3)


def _expert_u_body(xcm_ref, w1_ref, b1_ref, w2_ref, b2_ref, w3_ref,
                   b3_ref, o_ref):
    h1 = lax.dot_general(xcm_ref[:], w1_ref[:], (((0,), (0,)), ((), ())),
                         preferred_element_type=jnp.float32)
    h1 = jnp.maximum(h1 + b1_ref[:], 0.0)
    for e in range(NE):
        h2 = jnp.dot(h1[:, e * D_HID:(e + 1) * D_HID], w2_ref[e],
                     preferred_element_type=jnp.float32)
        h2 = jnp.maximum(h2 + b2_ref[:, e * D_HID:(e + 1) * D_HID], 0.0)
        u = jnp.dot(h2, w3_ref[e], preferred_element_type=jnp.float32)
        o_ref[e] = (u + b3_ref[e:e + 1]).astype(jnp.bfloat16)


def _expert_u(x_cm, W1, b1, W2, b2, W3, b3, base, nsteps):
    HID_ALL = NE * D_HID
    w1f = W1.transpose(1, 0, 2).reshape(D_IN, HID_ALL)
    b1f = b1.reshape(1, HID_ALL)
    b2f = b2.reshape(1, HID_ALL)
    return pl.pallas_call(
        _expert_u_body,
        grid=(nsteps,),
        in_specs=[
            pl.BlockSpec((D_IN, _T_F), lambda i: (0, base + i)),
            pl.BlockSpec((D_IN, HID_ALL), lambda i: (0, 0)),
            pl.BlockSpec((1, HID_ALL), lambda i: (0, 0)),
            pl.BlockSpec((NE, D_HID, D_HID), lambda i: (0, 0, 0)),
            pl.BlockSpec((1, HID_ALL), lambda i: (0, 0)),
            pl.BlockSpec((NE, D_HID, D_OUT), lambda i: (0, 0, 0)),
            pl.BlockSpec((NE, D_OUT), lambda i: (0, 0)),
        ],
        out_specs=pl.BlockSpec((NE, _T_F, D_OUT), lambda i: (0, i, 0)),
        out_shape=jax.ShapeDtypeStruct((NE, nsteps * _T_F, D_OUT),
                                       jnp.bfloat16),
    )(x_cm, w1f, b1f, W2, b2f, W3, b3)


def _combine_body(u0_ref, u1_ref, u2_ref, wn_ref, o_ref):
    i = pl.program_id(0)
    p = jnp.where(i < _PIECES[0], 0,
                  jnp.where(i < _PIECES[0] + _PIECES[1], 1, 2))
    u = lax.switch(p, [lambda: u0_ref[:], lambda: u1_ref[:],
                       lambda: u2_ref[:]])
    wn = wn_ref[:]
    y = jnp.zeros((_T_F, D_OUT), jnp.float32)
    for e in range(NE):
        y = y + u[e].astype(jnp.float32) * wn[:, e:e + 1]
    o_ref[:] = y


def _combine(u0, u1, u2, wnode):
    p0, p1, _ = _PIECES
    return pl.pallas_call(
        _combine_body,
        grid=(N_PAD // _T_F,),
        in_specs=[
            pl.BlockSpec((NE, _T_F, D_OUT),
                         lambda i: (0, jnp.clip(i, 0, _PIECES[0] - 1), 0)),
            pl.BlockSpec((NE, _T_F, D_OUT),
                         lambda i: (0, jnp.clip(i - p0, 0, _PIECES[1] - 1), 0)),
            pl.BlockSpec((NE, _T_F, D_OUT),
                         lambda i: (0, jnp.clip(i - p0 - p1, 0,
                                                _PIECES[2] - 1), 0)),
            pl.BlockSpec((_T_F, NE), lambda i: (i, 0)),
        ],
        out_specs=pl.BlockSpec((_T_F, D_OUT), lambda i: (i, 0)),
        out_shape=jax.ShapeDtypeStruct((N_PAD, D_OUT), jnp.float32),
    )(u0, u1, u2, wnode)


def kernel(atomic_numbers, pos, edge_index, batch, W1, b1, W2, b2, W3, b3,
           Wg1, bg1, Wg2, bg2):
    x_cm = jnp.pad(
        jnp.concatenate([atomic_numbers[None, :], pos.T], axis=0),
        ((0, 0), (0, N_PAD - N_NODES)))
    batch_pad = jnp.pad(batch, (0, N_PAD - N_NODES),
                        constant_values=N_GRAPHS)

    aggp = _edge_scatter(x_cm.reshape(-1), edge_index.reshape(-1))
    base = 0
    us = []
    for nsteps in _PIECES:
        us.append(_expert_u(x_cm, W1, b1, W2, b2, W3, b3, base, nsteps))
        base += nsteps
    hg = _gate_mlp(aggp.reshape(NW, D_IN, N_PAD), x_cm, Wg1, bg1)
    gp, cnt = _segsum_cnt(hg, batch_pad)
    wts = _gate_head(gp, cnt, Wg2, bg2)
    wnode = _wgather(wts.reshape(-1), batch_pad).reshape(N_PAD, NE)
    y = _combine(us[0], us[1], us[2], wnode)
    (out,) = _segsum_plain(y, batch_pad)
    return out


# speedup vs baseline: 1.2209x; 1.2209x over previous
import functools

import jax
import jax.numpy as jnp
from jax import lax
from jax.experimental import pallas as pl
from jax.experimental.pallas import tpu as pltpu
from jax.experimental.pallas import tpu_sc as plsc

N_NODES = 10000
N_EDGES = 160000
N_GRAPHS = 1024
NE = 8
D_IN = 4
D_HID = 512
D_OUT = 128
D_GATE = 128

NC, NS, L = 2, 16, 16
NW = NC * NS
N_PAD = 10240
ROWS_PER_W = N_PAD // NW
EDGES_PER_W = N_EDGES // NW
G_PER_W = N_GRAPHS // NW
BW_PAD = 1032
LOG2N = 14

_MESH = plsc.VectorSubcoreMesh(core_axis_name="c", subcore_axis_name="s",
                               num_cores=NC, num_subcores=NS)
_SC_PARAMS = pltpu.CompilerParams(needs_layout_passes=False)

_MM_DTYPE = jnp.float32


def _wid():
    return lax.axis_index("s") * NC + lax.axis_index("c")


@functools.partial(
    pl.kernel, mesh=_MESH, compiler_params=_SC_PARAMS,
    out_type=jax.ShapeDtypeStruct((NW, D_IN * N_PAD), jnp.float32),
    scratch_types=[
        pltpu.VMEM((D_IN * N_PAD,), jnp.float32),
        pltpu.VMEM((D_IN * N_PAD,), jnp.float32),
        pltpu.VMEM((EDGES_PER_W + 8,), jnp.int32),
        pltpu.VMEM((EDGES_PER_W + 8,), jnp.int32),
        pltpu.SemaphoreType.DMA,
        pltpu.SemaphoreType.DMA,
        pltpu.SemaphoreType.DMA,
    ],
)
def _edge_scatter(x_hbm, ei_hbm, aggp_hbm, xv, av, sv, dv, smx, sms, smd):
    w = _wid()
    dx = pltpu.make_async_copy(x_hbm, xv, smx)
    dx.start()
    ds = pltpu.make_async_copy(
        ei_hbm.at[pl.ds(w * EDGES_PER_W, EDGES_PER_W)],
        sv.at[pl.ds(0, EDGES_PER_W)], sms)
    ds.start()
    dd = pltpu.make_async_copy(
        ei_hbm.at[pl.ds(N_EDGES + w * EDGES_PER_W, EDGES_PER_W)],
        dv.at[pl.ds(0, EDGES_PER_W)], smd)
    dd.start()

    def zero(i, _):
        av[pl.ds(i * L, L)] = jnp.zeros((L,), jnp.float32)
        return 0
    lax.fori_loop(0, D_IN * N_PAD // L, zero, 0)
    dx.wait()
    ds.wait()
    dd.wait()

    def group(g, mask):
        s16 = sv[pl.ds(g * L, L)]
        d16 = dv[pl.ds(g * L, L)]
        for c in range(D_IN):
            vals = plsc.load_gather(xv, [s16 + c * N_PAD], mask=mask)
            plsc.addupdate_scatter(av, [d16 + c * N_PAD], vals, mask=mask)

    def body(g, _):
        group(g, None)
        return 0
    lax.fori_loop(0, EDGES_PER_W // L, body, 0)
    group(EDGES_PER_W // L, lax.iota(jnp.int32, L) < EDGES_PER_W % L)
    pltpu.sync_copy(av, aggp_hbm.at[w])


def _gate_mlp_body(aggp_ref, xcm_ref, wg1_ref, bg1_ref, o_ref):
    agg_cm = xcm_ref[:]
    for i in range(NW):
        agg_cm = agg_cm + aggp_ref[i]
    hg = lax.dot_general(agg_cm, wg1_ref[:], (((0,), (0,)), ((), ())),
                         preferred_element_type=jnp.float32)
    o_ref[:] = jnp.maximum(hg + bg1_ref[:], 0.0)


def _gate_mlp(aggp, x_cm, Wg1, bg1):
    return pl.pallas_call(
        _gate_mlp_body,
        out_shape=jax.ShapeDtypeStruct((N_PAD, D_GATE), jnp.float32),
    )(aggp, x_cm, Wg1, bg1.reshape(1, D_GATE))


def _lower_bound_vec(bv, targets):
    def step(_, lohi):
        lo, hi = lohi
        mid = (lo + hi) // 2
        vals = plsc.load_gather(bv, [mid])
        pred = vals < targets
        return (jnp.where(pred, mid + 1, lo), jnp.where(pred, hi, mid))
    zeros = jnp.zeros((L,), jnp.int32)
    lo, _ = lax.fori_loop(0, LOG2N, step, (zeros, zeros + N_PAD))
    return lo


def _lower_bound_scalar(bv, target):
    return _lower_bound_vec(bv, jnp.full((L,), target, jnp.int32))[0]


def _make_segsum(head):
    D = D_GATE
    if head:
        out_ty = [jax.ShapeDtypeStruct((BW_PAD * NE,), jnp.float32)]
    else:
        out_ty = [jax.ShapeDtypeStruct((N_GRAPHS, D), jnp.float32)]
    scratch = [
        pltpu.VMEM((N_PAD,), jnp.int32),
        pltpu.VMEM((G_PER_W, D), jnp.float32),
        pltpu.VMEM((L, D), jnp.float32),
        pltpu.VMEM((L, D), jnp.float32),
        pltpu.SemaphoreType.DMA,
        pltpu.SemaphoreType.DMA,
    ]
    if head:
        scratch += [
            pltpu.VMEM((G_PER_W,), jnp.float32),
            pltpu.VMEM((NE * D_GATE,), jnp.float32),
            pltpu.VMEM((L,), jnp.float32),
            pltpu.VMEM((G_PER_W * NE,), jnp.float32),
            pltpu.VMEM(((BW_PAD - N_GRAPHS) * NE,), jnp.float32),
        ]

    def body(rows_hbm, batch_hbm, *refs):
        if head:
            (wg2t_hbm, bg2_hbm, wts_hbm, bv, acc, rb0, rb1, sem0, sem1,
             cntv, wg2v, bg2v, wloc, zbuf) = refs
        else:
            gp_hbm, bv, acc, rb0, rb1, sem0, sem1 = refs
        w = _wid()
        g0 = w * G_PER_W
        pltpu.sync_copy(batch_hbm, bv)
        if head:
            pltpu.sync_copy(wg2t_hbm, wg2v)
            pltpu.sync_copy(bg2_hbm, bg2v)
        lo = _lower_bound_scalar(bv, g0)
        hi = _lower_bound_scalar(bv, g0 + G_PER_W)
        lo8 = pl.multiple_of((lo // 8) * 8, 8)
        nch = (hi - lo8 + L - 1) // L

        def chunk_base(cidx):
            r0 = jnp.minimum(lo8 + cidx * L, N_PAD - L)
            return pl.multiple_of(r0, 8)

        def start(cidx, rb, sem):
            pltpu.make_async_copy(
                rows_hbm.at[pl.ds(chunk_base(cidx), L)], rb, sem).start()

        def process(cidx, rb):
            r0 = chunk_base(cidx)
            bchunk = bv[pl.ds(r0, L)]
            for j in range(L):
                r = r0 + j
                tgt = bchunk[j] - g0

                @pl.when((r >= lo) & (r < hi))
                def _():
                    for v in range(D // L):
                        plsc.addupdate(acc.at[tgt, pl.ds(v * L, L)],
                                       rb[j, pl.ds(v * L, L)])

        @pl.when(nch > 0)
        def _():
            start(0, rb0, sem0)

        for gl in range(G_PER_W):
            for v in range(D // L):
                acc[gl, pl.ds(v * L, L)] = jnp.zeros((L,), jnp.float32)

        def pair(m, _):
            k = m * 2

            @pl.when(k + 1 < nch)
            def _():
                start(k + 1, rb1, sem1)

            @pl.when(k < nch)
            def _():
                pltpu.make_async_copy(
                    rows_hbm.at[pl.ds(0, L)], rb0, sem0).wait()
                process(k, rb0)

            @pl.when(k + 2 < nch)
            def _():
                start(k + 2, rb0, sem0)

            @pl.when(k + 1 < nch)
            def _():
                pltpu.make_async_copy(
                    rows_hbm.at[pl.ds(0, L)], rb1, sem1).wait()
                process(k + 1, rb1)
            return 0
        lax.fori_loop(0, (nch + 1) // 2, pair, 0)

        iota = lax.iota(jnp.int32, L)
        if not head:
            pltpu.sync_copy(acc, gp_hbm.at[pl.ds(g0, G_PER_W)])
            return

        for h in range(G_PER_W // L):
            t0 = g0 + h * L + iota
            b_lo = _lower_bound_vec(bv, t0)
            b_hi = _lower_bound_vec(bv, t0 + 1)
            cntv[pl.ds(h * L, L)] = (b_hi - b_lo).astype(jnp.float32)

        bg2vec = bg2v[pl.ds(0, L)]
        for h in range(G_PER_W // L):
            g16l = h * L + iota
            cnt16 = cntv[pl.ds(h * L, L)]
            scale = 1.0 / jnp.maximum(cnt16, 1.0)

            def dgstep(dg, ls):
                new = list(ls)
                wvs = [wg2v[pl.ds(e * D_GATE + dg * L, L)]
                       for e in range(NE)]
                for dl in range(L):
                    col = plsc.load_gather(
                        acc, [g16l, jnp.full((L,), dg * L + dl)])
                    for e in range(NE):
                        new[e] = new[e] + col * wvs[e][dl]
                return tuple(new)
            ls = lax.fori_loop(0, D_GATE // L, dgstep,
                               tuple(jnp.zeros((L,), jnp.float32)
                                     for _ in range(NE)))
            logit = [ls[e] * scale + bg2vec[e] for e in range(NE)]
            m = logit[0]
            for e in range(1, NE):
                m = jnp.maximum(m, logit[e])
            ex = [jnp.exp(logit[e] - m) for e in range(NE)]
            tot = ex[0]
            for e in range(1, NE):
                tot = tot + ex[e]
            winv = 1.0 / tot
            for e in range(NE):
                plsc.store_scatter(wloc, [g16l * NE + e], ex[e] * winv)
        pltpu.sync_copy(wloc, wts_hbm.at[pl.ds(g0 * NE, G_PER_W * NE)])

        @pl.when(w == NW - 1)
        def _():
            npadw = (BW_PAD - N_GRAPHS) * NE
            for i in range(npadw // L):
                zbuf[pl.ds(i * L, L)] = jnp.zeros((L,), jnp.float32)
            pltpu.sync_copy(zbuf, wts_hbm.at[pl.ds(N_GRAPHS * NE, npadw)])

    return pl.kernel(body, mesh=_MESH, compiler_params=_SC_PARAMS,
                     out_type=tuple(out_ty), scratch_types=scratch)


_segsum_head = _make_segsum(True)
_segsum_plain = _make_segsum(False)


@functools.partial(
    pl.kernel, mesh=_MESH, compiler_params=_SC_PARAMS,
    out_type=jax.ShapeDtypeStruct((N_PAD * NE,), jnp.float32),
    scratch_types=[
        pltpu.VMEM((BW_PAD * NE,), jnp.float32),
        pltpu.VMEM((ROWS_PER_W,), jnp.int32),
        pltpu.VMEM((ROWS_PER_W * NE,), jnp.float32),
    ],
)
def _wgather(wt_hbm, batch_hbm, out_hbm, wt, bvt, ov):
    w = _wid()
    pltpu.sync_copy(wt_hbm, wt)
    pltpu.sync_copy(batch_hbm.at[pl.ds(w * ROWS_PER_W, ROWS_PER_W)], bvt)
    iota = lax.iota(jnp.int32, L)

    def body(k, _):
        idx = bvt[pl.ds(k * L, L)] * NE
        for c in range(NE):
            vals = plsc.load_gather(wt, [idx + c])
            plsc.store_scatter(ov, [iota * NE + (k * L * NE + c)], vals)
        return 0
    lax.fori_loop(0, ROWS_PER_W // L, body, 0)
    pltpu.sync_copy(ov, out_hbm.at[pl.ds(w * ROWS_PER_W * NE, ROWS_PER_W * NE)])


def _expert_body(xcm_ref, wn_ref, w1_ref, b1_ref, w2_ref, b2_ref, w3_ref,
                 b3_ref, o_ref):
    h1 = lax.dot_general(xcm_ref[:], w1_ref[:], (((0,), (0,)), ((), ())),
                         preferred_element_type=jnp.float32)
    h1 = jnp.maximum(h1 + b1_ref[:], 0.0)
    wn = wn_ref[:]
    y = jnp.dot(wn, b3_ref[:], preferred_element_type=jnp.float32)
    for e in range(NE):
        h2 = jnp.dot(h1[:, e * D_HID:(e + 1) * D_HID], w2_ref[e],
                     preferred_element_type=jnp.float32)
        h2 = jnp.maximum(h2 + b2_ref[:, e * D_HID:(e + 1) * D_HID], 0.0)
        u = jnp.dot(h2, w3_ref[e], preferred_element_type=jnp.float32)
        y = y + u * wn[:, e:e + 1]
    o_ref[:] = y


def _expert_mlp(x_cm, wnode, W1, b1, W2, b2, W3, b3):
    T = 1024
    grid = (N_PAD // T,)
    HID_ALL = NE * D_HID
    w1f = W1.transpose(1, 0, 2).reshape(D_IN, HID_ALL)
    b1f = b1.reshape(1, HID_ALL)
    b2f = b2.reshape(1, HID_ALL)
    return pl.pallas_call(
        _expert_body,
        grid=grid,
        in_specs=[
            pl.BlockSpec((D_IN, T), lambda i: (0, i)),
            pl.BlockSpec((T, NE), lambda i: (i, 0)),
            pl.BlockSpec((D_IN, HID_ALL), lambda i: (0, 0)),
            pl.BlockSpec((1, HID_ALL), lambda i: (0, 0)),
            pl.BlockSpec((NE, D_HID, D_HID), lambda i: (0, 0, 0)),
            pl.BlockSpec((1, HID_ALL), lambda i: (0, 0)),
            pl.BlockSpec((NE, D_HID, D_OUT), lambda i: (0, 0, 0)),
            pl.BlockSpec((NE, D_OUT), lambda i: (0, 0)),
        ],
        out_specs=pl.BlockSpec((T, D_OUT), lambda i: (i, 0)),
        out_shape=jax.ShapeDtypeStruct((N_PAD, D_OUT), jnp.float32),
    )(x_cm, wnode, w1f, b1f, W2, b2f, W3, b3)


def kernel(atomic_numbers, pos, edge_index, batch, W1, b1, W2, b2, W3, b3,
           Wg1, bg1, Wg2, bg2):
    x_cm = jnp.pad(
        jnp.concatenate([atomic_numbers[None, :], pos.T], axis=0),
        ((0, 0), (0, N_PAD - N_NODES)))
    batch_pad = jnp.pad(batch, (0, N_PAD - N_NODES),
                        constant_values=N_GRAPHS)

    aggp = _edge_scatter(x_cm.reshape(-1), edge_index.reshape(-1))
    hg = _gate_mlp(aggp.reshape(NW, D_IN, N_PAD), x_cm, Wg1, bg1)
    (wts_flat,) = _segsum_head(hg, batch_pad, Wg2.T.reshape(-1),
                               jnp.pad(bg2, (0, L - NE)))
    wnode = _wgather(wts_flat, batch_pad).reshape(N_PAD, NE)
    y = _expert_mlp(x_cm, wnode, W1, b1, W2, b2, W3, b3)
    (out,) = _segsum_plain(y, batch_pad)
    return out


# speedup vs baseline: 1.2383x; 1.0142x over previous
import functools

import jax
import jax.numpy as jnp
from jax import lax
from jax.experimental import pallas as pl
from jax.experimental.pallas import tpu as pltpu
from jax.experimental.pallas import tpu_sc as plsc

N_NODES = 10000
N_EDGES = 160000
N_GRAPHS = 1024
NE = 8
D_IN = 4
D_HID = 512
D_OUT = 128
D_GATE = 128

NC, NS, L = 2, 16, 16
NW = NC * NS
N_PAD = 10240
ROWS_PER_W = N_PAD // NW
EDGES_PER_W = N_EDGES // NW
G_PER_W = N_GRAPHS // NW
BW_PAD = 1032
LOG2N = 14

_MESH = plsc.VectorSubcoreMesh(core_axis_name="c", subcore_axis_name="s",
                               num_cores=NC, num_subcores=NS)
_SC_PARAMS = pltpu.CompilerParams(needs_layout_passes=False)

_MM_DTYPE = jnp.float32


def _wid():
    return lax.axis_index("s") * NC + lax.axis_index("c")


@functools.partial(
    pl.kernel, mesh=_MESH, compiler_params=_SC_PARAMS,
    out_type=jax.ShapeDtypeStruct((NW, D_IN * N_PAD), jnp.float32),
    scratch_types=[
        pltpu.VMEM((D_IN * N_PAD,), jnp.float32),
        pltpu.VMEM((D_IN * N_PAD,), jnp.float32),
        pltpu.VMEM((EDGES_PER_W + 8,), jnp.int32),
        pltpu.VMEM((EDGES_PER_W + 8,), jnp.int32),
        pltpu.SemaphoreType.DMA,
        pltpu.SemaphoreType.DMA,
        pltpu.SemaphoreType.DMA,
    ],
)
def _edge_scatter(x_hbm, ei_hbm, aggp_hbm, xv, av, sv, dv, smx, sms, smd):
    w = _wid()
    dx = pltpu.make_async_copy(x_hbm, xv, smx)
    dx.start()
    ds = pltpu.make_async_copy(
        ei_hbm.at[pl.ds(w * EDGES_PER_W, EDGES_PER_W)],
        sv.at[pl.ds(0, EDGES_PER_W)], sms)
    ds.start()
    dd = pltpu.make_async_copy(
        ei_hbm.at[pl.ds(N_EDGES + w * EDGES_PER_W, EDGES_PER_W)],
        dv.at[pl.ds(0, EDGES_PER_W)], smd)
    dd.start()

    def zero(i, _):
        av[pl.ds(i * L, L)] = jnp.zeros((L,), jnp.float32)
        return 0
    lax.fori_loop(0, D_IN * N_PAD // L, zero, 0)
    dx.wait()
    ds.wait()
    dd.wait()

    def group(g, mask):
        s16 = sv[pl.ds(g * L, L)]
        d16 = dv[pl.ds(g * L, L)]
        for c in range(D_IN):
            vals = plsc.load_gather(xv, [s16 + c * N_PAD], mask=mask)
            plsc.addupdate_scatter(av, [d16 + c * N_PAD], vals, mask=mask)

    def body(g, _):
        group(g, None)
        return 0
    lax.fori_loop(0, EDGES_PER_W // L, body, 0)
    group(EDGES_PER_W // L, lax.iota(jnp.int32, L) < EDGES_PER_W % L)
    pltpu.sync_copy(av, aggp_hbm.at[w])


def _gate_mlp_body(aggp_ref, xcm_ref, wg1_ref, bg1_ref, o_ref):
    agg_cm = xcm_ref[:]
    for i in range(NW):
        agg_cm = agg_cm + aggp_ref[i]
    hg = lax.dot_general(agg_cm, wg1_ref[:], (((0,), (0,)), ((), ())),
                         preferred_element_type=jnp.float32)
    o_ref[:] = jnp.maximum(hg + bg1_ref[:], 0.0)


def _gate_mlp(aggp, x_cm, Wg1, bg1):
    return pl.pallas_call(
        _gate_mlp_body,
        out_shape=jax.ShapeDtypeStruct((N_PAD, D_GATE), jnp.float32),
    )(aggp, x_cm, Wg1, bg1.reshape(1, D_GATE))


def _lower_bound_vec(bv, targets):
    def step(_, lohi):
        lo, hi = lohi
        mid = (lo + hi) // 2
        vals = plsc.load_gather(bv, [mid])
        pred = vals < targets
        return (jnp.where(pred, mid + 1, lo), jnp.where(pred, hi, mid))
    zeros = jnp.zeros((L,), jnp.int32)
    lo, _ = lax.fori_loop(0, LOG2N, step, (zeros, zeros + N_PAD))
    return lo


def _lower_bound_scalar(bv, target):
    return _lower_bound_vec(bv, jnp.full((L,), target, jnp.int32))[0]


def _make_segsum(with_cnt):
    D = D_GATE
    out_ty = [jax.ShapeDtypeStruct((N_GRAPHS, D), jnp.float32)]
    if with_cnt:
        out_ty.append(jax.ShapeDtypeStruct((N_GRAPHS,), jnp.float32))
    scratch = [
        pltpu.VMEM((N_PAD,), jnp.int32),
        pltpu.VMEM((G_PER_W, D), jnp.float32),
        pltpu.VMEM((L, D), jnp.float32),
        pltpu.VMEM((L, D), jnp.float32),
        pltpu.SemaphoreType.DMA,
        pltpu.SemaphoreType.DMA,
    ]
    if with_cnt:
        scratch.append(pltpu.VMEM((G_PER_W,), jnp.float32))

    def body(rows_hbm, batch_hbm, *refs):
        if with_cnt:
            gp_hbm, cnt_hbm, bv, acc, rb0, rb1, sem0, sem1, cntv = refs
        else:
            gp_hbm, bv, acc, rb0, rb1, sem0, sem1 = refs
        w = _wid()
        g0 = w * G_PER_W
        pltpu.sync_copy(batch_hbm, bv)
        lo = _lower_bound_scalar(bv, g0)
        hi = _lower_bound_scalar(bv, g0 + G_PER_W)
        lo8 = pl.multiple_of((lo // 8) * 8, 8)
        nch = (hi - lo8 + L - 1) // L

        def chunk_base(cidx):
            r0 = jnp.minimum(lo8 + cidx * L, N_PAD - L)
            return pl.multiple_of(r0, 8)

        def start(cidx, rb, sem):
            pltpu.make_async_copy(
                rows_hbm.at[pl.ds(chunk_base(cidx), L)], rb, sem).start()

        def process(cidx, rb):
            r0 = chunk_base(cidx)
            bchunk = bv[pl.ds(r0, L)]
            for j in range(L):
                r = r0 + j
                tgt = bchunk[j] - g0

                @pl.when((r >= lo) & (r < hi))
                def _():
                    for v in range(D // L):
                        plsc.addupdate(acc.at[tgt, pl.ds(v * L, L)],
                                       rb[j, pl.ds(v * L, L)])

        @pl.when(nch > 0)
        def _():
            start(0, rb0, sem0)

        for gl in range(G_PER_W):
            for v in range(D // L):
                acc[gl, pl.ds(v * L, L)] = jnp.zeros((L,), jnp.float32)

        def pair(m, _):
            k = m * 2

            @pl.when(k + 1 < nch)
            def _():
                start(k + 1, rb1, sem1)

            @pl.when(k < nch)
            def _():
                pltpu.make_async_copy(
                    rows_hbm.at[pl.ds(0, L)], rb0, sem0).wait()
                process(k, rb0)

            @pl.when(k + 2 < nch)
            def _():
                start(k + 2, rb0, sem0)

            @pl.when(k + 1 < nch)
            def _():
                pltpu.make_async_copy(
                    rows_hbm.at[pl.ds(0, L)], rb1, sem1).wait()
                process(k + 1, rb1)
            return 0
        lax.fori_loop(0, (nch + 1) // 2, pair, 0)
        pltpu.sync_copy(acc, gp_hbm.at[pl.ds(g0, G_PER_W)])

        if with_cnt:
            iota = lax.iota(jnp.int32, L)
            for h in range(G_PER_W // L):
                t0 = g0 + h * L + iota
                b_lo = _lower_bound_vec(bv, t0)
                b_hi = _lower_bound_vec(bv, t0 + 1)
                cntv[pl.ds(h * L, L)] = (b_hi - b_lo).astype(jnp.float32)
            pltpu.sync_copy(cntv, cnt_hbm.at[pl.ds(g0, G_PER_W)])

    return pl.kernel(body, mesh=_MESH, compiler_params=_SC_PARAMS,
                     out_type=tuple(out_ty), scratch_types=scratch)


_segsum_cnt = _make_segsum(True)
_segsum_plain = _make_segsum(False)


def _gate_head_body(gp_ref, cnt_ref, wg2_ref, bg2_ref, o_ref):
    cnt = jnp.maximum(cnt_ref[:], 1.0)
    gp = gp_ref[:] / cnt
    logits = jnp.dot(gp, wg2_ref[:], preferred_element_type=jnp.float32)
    logits = logits + bg2_ref[:]
    m = jnp.max(logits, axis=1, keepdims=True)
    ex = jnp.exp(logits - m)
    wts = ex / jnp.sum(ex, axis=1, keepdims=True)
    o_ref[:N_GRAPHS] = wts
    o_ref[N_GRAPHS:] = jnp.zeros((BW_PAD - N_GRAPHS, NE), jnp.float32)


def _gate_head(gp, cnt, Wg2, bg2):
    return pl.pallas_call(
        _gate_head_body,
        out_shape=jax.ShapeDtypeStruct((BW_PAD, NE), jnp.float32),
    )(gp, cnt.reshape(N_GRAPHS, 1), Wg2, bg2.reshape(1, NE))


@functools.partial(
    pl.kernel, mesh=_MESH, compiler_params=_SC_PARAMS,
    out_type=jax.ShapeDtypeStruct((N_PAD * NE,), jnp.float32),
    scratch_types=[
        pltpu.VMEM((BW_PAD * NE,), jnp.float32),
        pltpu.VMEM((ROWS_PER_W,), jnp.int32),
        pltpu.VMEM((ROWS_PER_W * NE,), jnp.float32),
    ],
)
def _wgather(wt_hbm, batch_hbm, out_hbm, wt, bvt, ov):
    w = _wid()
    pltpu.sync_copy(wt_hbm, wt)
    pltpu.sync_copy(batch_hbm.at[pl.ds(w * ROWS_PER_W, ROWS_PER_W)], bvt)
    iota = lax.iota(jnp.int32, L)

    def body(k, _):
        idx = bvt[pl.ds(k * L, L)] * NE
        for c in range(NE):
            vals = plsc.load_gather(wt, [idx + c])
            plsc.store_scatter(ov, [iota * NE + (k * L * NE + c)], vals)
        return 0
    lax.fori_loop(0, ROWS_PER_W // L, body, 0)
    pltpu.sync_copy(ov, out_hbm.at[pl.ds(w * ROWS_PER_W * NE, ROWS_PER_W * NE)])


def _expert_body(xcm_ref, wn_ref, w1_ref, b1_ref, w2_ref, b2_ref, w3_ref,
                 b3_ref, o_ref):
    h1 = lax.dot_general(xcm_ref[:], w1_ref[:], (((0,), (0,)), ((), ())),
                         preferred_element_type=jnp.float32)
    h1 = jnp.maximum(h1 + b1_ref[:], 0.0)
    wn = wn_ref[:]
    y = jnp.dot(wn, b3_ref[:], preferred_element_type=jnp.float32)
    for e in range(NE):
        h2 = jnp.dot(h1[:, e * D_HID:(e + 1) * D_HID], w2_ref[e],
                     preferred_element_type=jnp.float32)
        h2 = jnp.maximum(h2 + b2_ref[:, e * D_HID:(e + 1) * D_HID], 0.0)
        u = jnp.dot(h2, w3_ref[e], preferred_element_type=jnp.float32)
        y = y + u * wn[:, e:e + 1]
    o_ref[:] = y


def _expert_mlp(x_cm, wnode, W1, b1, W2, b2, W3, b3):
    T = 1024
    grid = (N_PAD // T,)
    HID_ALL = NE * D_HID
    w1f = W1.transpose(1, 0, 2).reshape(D_IN, HID_ALL)
    b1f = b1.reshape(1, HID_ALL)
    b2f = b2.reshape(1, HID_ALL)
    return pl.pallas_call(
        _expert_body,
        grid=grid,
        in_specs=[
            pl.BlockSpec((D_IN, T), lambda i: (0, i)),
            pl.BlockSpec((T, NE), lambda i: (i, 0)),
            pl.BlockSpec((D_IN, HID_ALL), lambda i: (0, 0)),
            pl.BlockSpec((1, HID_ALL), lambda i: (0, 0)),
            pl.BlockSpec((NE, D_HID, D_HID), lambda i: (0, 0, 0)),
            pl.BlockSpec((1, HID_ALL), lambda i: (0, 0)),
            pl.BlockSpec((NE, D_HID, D_OUT), lambda i: (0, 0, 0)),
            pl.BlockSpec((NE, D_OUT), lambda i: (0, 0)),
        ],
        out_specs=pl.BlockSpec((T, D_OUT), lambda i: (i, 0)),
        out_shape=jax.ShapeDtypeStruct((N_PAD, D_OUT), jnp.float32),
    )(x_cm, wnode, w1f, b1f, W2, b2f, W3, b3)


def kernel(atomic_numbers, pos, edge_index, batch, W1, b1, W2, b2, W3, b3,
           Wg1, bg1, Wg2, bg2):
    x_cm = jnp.pad(
        jnp.concatenate([atomic_numbers[None, :], pos.T], axis=0),
        ((0, 0), (0, N_PAD - N_NODES)))
    batch_pad = jnp.pad(batch, (0, N_PAD - N_NODES),
                        constant_values=N_GRAPHS)

    aggp = _edge_scatter(x_cm.reshape(-1), edge_index.reshape(-1))
    hg = _gate_mlp(aggp.reshape(NW, D_IN, N_PAD), x_cm, Wg1, bg1)
    gp, cnt = _segsum_cnt(hg, batch_pad)
    wts = _gate_head(gp, cnt, Wg2, bg2)
    wnode = _wgather(wts.reshape(-1), batch_pad).reshape(N_PAD, NE)
    y = _expert_mlp(x_cm, wnode, W1, b1, W2, b2, W3, b3)
    (out,) = _segsum_plain(y, batch_pad)
    return out


# speedup vs baseline: 1.2802x; 1.0339x over previous
import functools

import jax
import jax.numpy as jnp
from jax import lax
from jax.experimental import pallas as pl
from jax.experimental.pallas import tpu as pltpu
from jax.experimental.pallas import tpu_sc as plsc

N_NODES = 10000
N_EDGES = 160000
N_GRAPHS = 1024
NE = 8
D_IN = 4
D_HID = 512
D_OUT = 128
D_GATE = 128

NC, NS, L = 2, 16, 16
NW = NC * NS
N_PAD = 10240
ROWS_PER_W = N_PAD // NW
EDGES_PER_W = N_EDGES // NW
G_PER_W = N_GRAPHS // NW
BW_PAD = 1032
LOG2N = 14

_MESH = plsc.VectorSubcoreMesh(core_axis_name="c", subcore_axis_name="s",
                               num_cores=NC, num_subcores=NS)
_SC_PARAMS = pltpu.CompilerParams(needs_layout_passes=False)

_MM_DTYPE = jnp.float32


def _wid():
    return lax.axis_index("s") * NC + lax.axis_index("c")


@functools.partial(
    pl.kernel, mesh=_MESH, compiler_params=_SC_PARAMS,
    out_type=jax.ShapeDtypeStruct((NW, D_IN * N_PAD), jnp.float32),
    scratch_types=[
        pltpu.VMEM((D_IN * N_PAD,), jnp.float32),
        pltpu.VMEM((D_IN * N_PAD,), jnp.float32),
        pltpu.VMEM((EDGES_PER_W + 8,), jnp.int32),
        pltpu.VMEM((EDGES_PER_W + 8,), jnp.int32),
        pltpu.SemaphoreType.DMA,
        pltpu.SemaphoreType.DMA,
        pltpu.SemaphoreType.DMA,
    ],
)
def _edge_scatter(x_hbm, ei_hbm, aggp_hbm, xv, av, sv, dv, smx, sms, smd):
    w = _wid()
    dx = pltpu.make_async_copy(x_hbm, xv, smx)
    dx.start()
    ds = pltpu.make_async_copy(
        ei_hbm.at[pl.ds(w * EDGES_PER_W, EDGES_PER_W)],
        sv.at[pl.ds(0, EDGES_PER_W)], sms)
    ds.start()
    dd = pltpu.make_async_copy(
        ei_hbm.at[pl.ds(N_EDGES + w * EDGES_PER_W, EDGES_PER_W)],
        dv.at[pl.ds(0, EDGES_PER_W)], smd)
    dd.start()

    def zero(i, _):
        av[pl.ds(i * L, L)] = jnp.zeros((L,), jnp.float32)
        return 0
    lax.fori_loop(0, D_IN * N_PAD // L, zero, 0)
    dx.wait()
    ds.wait()
    dd.wait()

    def group(g, mask):
        s16 = sv[pl.ds(g * L, L)]
        d16 = dv[pl.ds(g * L, L)]
        for c in range(D_IN):
            vals = plsc.load_gather(xv, [s16 + c * N_PAD], mask=mask)
            plsc.addupdate_scatter(av, [d16 + c * N_PAD], vals, mask=mask)

    def body(g, _):
        group(g, None)
        return 0
    lax.fori_loop(0, EDGES_PER_W // L, body, 0)
    group(EDGES_PER_W // L, lax.iota(jnp.int32, L) < EDGES_PER_W % L)
    pltpu.sync_copy(av, aggp_hbm.at[w])


def _gate_mlp_body(aggp_ref, xcm_ref, wg1_ref, bg1_ref, o_ref):
    agg_flat = aggp_ref[0]
    for i in range(1, NW):
        agg_flat = agg_flat + aggp_ref[i]
    agg_cm = xcm_ref[:] + agg_flat.reshape(D_IN, N_PAD)
    hg = lax.dot_general(agg_cm, wg1_ref[:], (((0,), (0,)), ((), ())),
                         preferred_element_type=jnp.float32)
    o_ref[:] = jnp.maximum(hg + bg1_ref[:], 0.0)


def _gate_mlp(aggp, x_cm, Wg1, bg1):
    return pl.pallas_call(
        _gate_mlp_body,
        out_shape=jax.ShapeDtypeStruct((N_PAD, D_GATE), jnp.float32),
    )(aggp, x_cm, Wg1, bg1.reshape(1, D_GATE))


def _lower_bound_vec(bv, targets):
    def step(_, lohi):
        lo, hi = lohi
        mid = (lo + hi) // 2
        vals = plsc.load_gather(bv, [mid])
        pred = vals < targets
        return (jnp.where(pred, mid + 1, lo), jnp.where(pred, hi, mid))
    zeros = jnp.zeros((L,), jnp.int32)
    lo, _ = lax.fori_loop(0, LOG2N, step, (zeros, zeros + N_PAD))
    return lo


def _lower_bound_scalar(bv, target):
    return _lower_bound_vec(bv, jnp.full((L,), target, jnp.int32))[0]


def _make_segsum(with_cnt):
    D = D_GATE
    out_ty = [jax.ShapeDtypeStruct((N_GRAPHS, D), jnp.float32)]
    if with_cnt:
        out_ty.append(jax.ShapeDtypeStruct((N_GRAPHS,), jnp.float32))
    scratch = [
        pltpu.VMEM((N_PAD,), jnp.int32),
        pltpu.VMEM((G_PER_W, D), jnp.float32),
        pltpu.VMEM((L, D), jnp.float32),
        pltpu.VMEM((L, D), jnp.float32),
        pltpu.SemaphoreType.DMA,
        pltpu.SemaphoreType.DMA,
    ]
    if with_cnt:
        scratch.append(pltpu.VMEM((G_PER_W,), jnp.float32))

    def body(rows_hbm, batch_hbm, *refs):
        if with_cnt:
            gp_hbm, cnt_hbm, bv, acc, rb0, rb1, sem0, sem1, cntv = refs
        else:
            gp_hbm, bv, acc, rb0, rb1, sem0, sem1 = refs
        w = _wid()
        g0 = w * G_PER_W
        pltpu.sync_copy(batch_hbm, bv)
        lo = _lower_bound_scalar(bv, g0)
        hi = _lower_bound_scalar(bv, g0 + G_PER_W)
        lo8 = pl.multiple_of((lo // 8) * 8, 8)
        nch = (hi - lo8 + L - 1) // L

        def chunk_base(cidx):
            r0 = jnp.minimum(lo8 + cidx * L, N_PAD - L)
            return pl.multiple_of(r0, 8)

        def start(cidx, rb, sem):
            pltpu.make_async_copy(
                rows_hbm.at[pl.ds(chunk_base(cidx), L)], rb, sem).start()

        def process(cidx, rb):
            r0 = chunk_base(cidx)
            bchunk = bv[pl.ds(r0, L)]
            for j in range(L):
                r = r0 + j
                tgt = bchunk[j] - g0

                @pl.when((r >= lo) & (r < hi))
                def _():
                    for v in range(D // L):
                        plsc.addupdate(acc.at[tgt, pl.ds(v * L, L)],
                                       rb[j, pl.ds(v * L, L)])

        @pl.when(nch > 0)
        def _():
            start(0, rb0, sem0)

        for gl in range(G_PER_W):
            for v in range(D // L):
                acc[gl, pl.ds(v * L, L)] = jnp.zeros((L,), jnp.float32)

        def pair(m, _):
            k = m * 2

            @pl.when(k + 1 < nch)
            def _():
                start(k + 1, rb1, sem1)

            @pl.when(k < nch)
            def _():
                pltpu.make_async_copy(
                    rows_hbm.at[pl.ds(0, L)], rb0, sem0).wait()
                process(k, rb0)

            @pl.when(k + 2 < nch)
            def _():
                start(k + 2, rb0, sem0)

            @pl.when(k + 1 < nch)
            def _():
                pltpu.make_async_copy(
                    rows_hbm.at[pl.ds(0, L)], rb1, sem1).wait()
                process(k + 1, rb1)
            return 0
        lax.fori_loop(0, (nch + 1) // 2, pair, 0)
        pltpu.sync_copy(acc, gp_hbm.at[pl.ds(g0, G_PER_W)])

        if with_cnt:
            iota = lax.iota(jnp.int32, L)
            for h in range(G_PER_W // L):
                t0 = g0 + h * L + iota
                b_lo = _lower_bound_vec(bv, t0)
                b_hi = _lower_bound_vec(bv, t0 + 1)
                cntv[pl.ds(h * L, L)] = (b_hi - b_lo).astype(jnp.float32)
            pltpu.sync_copy(cntv, cnt_hbm.at[pl.ds(g0, G_PER_W)])

    return pl.kernel(body, mesh=_MESH, compiler_params=_SC_PARAMS,
                     out_type=tuple(out_ty), scratch_types=scratch)


_segsum_cnt = _make_segsum(True)
_segsum_plain = _make_segsum(False)


def _gate_head_body(gp_ref, cnt_ref, wg2_ref, bg2_ref, o_ref):
    cnt = jnp.maximum(cnt_ref[:], 1.0)
    gp = gp_ref[:] / cnt
    logits = jnp.dot(gp, wg2_ref[:], preferred_element_type=jnp.float32)
    logits = logits + bg2_ref[:]
    m = jnp.max(logits, axis=1, keepdims=True)
    ex = jnp.exp(logits - m)
    wts = ex / jnp.sum(ex, axis=1, keepdims=True)
    o_ref[:N_GRAPHS] = wts
    o_ref[N_GRAPHS:] = jnp.zeros((BW_PAD - N_GRAPHS, NE), jnp.float32)


def _gate_head(gp, cnt, Wg2, bg2):
    return pl.pallas_call(
        _gate_head_body,
        out_shape=jax.ShapeDtypeStruct((BW_PAD, NE), jnp.float32),
    )(gp, cnt.reshape(N_GRAPHS, 1), Wg2, bg2.reshape(1, NE))


@functools.partial(
    pl.kernel, mesh=_MESH, compiler_params=_SC_PARAMS,
    out_type=jax.ShapeDtypeStruct((N_PAD * NE,), jnp.float32),
    scratch_types=[
        pltpu.VMEM((BW_PAD * NE,), jnp.float32),
        pltpu.VMEM((ROWS_PER_W,), jnp.int32),
        pltpu.VMEM((ROWS_PER_W * NE,), jnp.float32),
    ],
)
def _wgather(wt_hbm, batch_hbm, out_hbm, wt, bvt, ov):
    w = _wid()
    pltpu.sync_copy(wt_hbm, wt)
    pltpu.sync_copy(batch_hbm.at[pl.ds(w * ROWS_PER_W, ROWS_PER_W)], bvt)
    iota = lax.iota(jnp.int32, L)

    def body(k, _):
        idx = bvt[pl.ds(k * L, L)] * NE
        for c in range(NE):
            vals = plsc.load_gather(wt, [idx + c])
            plsc.store_scatter(ov, [iota * NE + (k * L * NE + c)], vals)
        return 0
    lax.fori_loop(0, ROWS_PER_W // L, body, 0)
    pltpu.sync_copy(ov, out_hbm.at[pl.ds(w * ROWS_PER_W * NE, ROWS_PER_W * NE)])


def _expert_body(xcm_ref, wn_ref, w1_ref, b1_ref, w2_ref, b2_ref, w3_ref,
                 b3_ref, o_ref):
    h1 = lax.dot_general(xcm_ref[:], w1_ref[:], (((0,), (0,)), ((), ())),
                         preferred_element_type=jnp.float32)
    h1 = jnp.maximum(h1 + b1_ref[:], 0.0)
    wn = wn_ref[:]
    y = jnp.dot(wn, b3_ref[:], preferred_element_type=jnp.float32)
    for e in range(NE):
        h2 = jnp.dot(h1[:, e * D_HID:(e + 1) * D_HID], w2_ref[e],
                     preferred_element_type=jnp.float32)
        h2 = jnp.maximum(h2 + b2_ref[:, e * D_HID:(e + 1) * D_HID], 0.0)
        u = jnp.dot(h2, w3_ref[e], preferred_element_type=jnp.float32)
        y = y + u * wn[:, e:e + 1]
    o_ref[:] = y


def _expert_mlp(x_cm, wnode, W1, b1, W2, b2, W3, b3):
    T = 1024
    grid = (N_PAD // T,)
    HID_ALL = NE * D_HID
    w1f = W1.transpose(1, 0, 2).reshape(D_IN, HID_ALL)
    b1f = b1.reshape(1, HID_ALL)
    b2f = b2.reshape(1, HID_ALL)
    return pl.pallas_call(
        _expert_body,
        grid=grid,
        in_specs=[
            pl.BlockSpec((D_IN, T), lambda i: (0, i)),
            pl.BlockSpec((T, NE), lambda i: (i, 0)),
            pl.BlockSpec((D_IN, HID_ALL), lambda i: (0, 0)),
            pl.BlockSpec((1, HID_ALL), lambda i: (0, 0)),
            pl.BlockSpec((NE, D_HID, D_HID), lambda i: (0, 0, 0)),
            pl.BlockSpec((1, HID_ALL), lambda i: (0, 0)),
            pl.BlockSpec((NE, D_HID, D_OUT), lambda i: (0, 0, 0)),
            pl.BlockSpec((NE, D_OUT), lambda i: (0, 0)),
        ],
        out_specs=pl.BlockSpec((T, D_OUT), lambda i: (i, 0)),
        out_shape=jax.ShapeDtypeStruct((N_PAD, D_OUT), jnp.float32),
    )(x_cm, wnode, w1f, b1f, W2, b2f, W3, b3)


def kernel(atomic_numbers, pos, edge_index, batch, W1, b1, W2, b2, W3, b3,
           Wg1, bg1, Wg2, bg2):
    x_cm = jnp.pad(
        jnp.concatenate([atomic_numbers[None, :], pos.T], axis=0),
        ((0, 0), (0, N_PAD - N_NODES)))
    batch_pad = jnp.pad(batch, (0, N_PAD - N_NODES),
                        constant_values=N_GRAPHS)

    aggp = _edge_scatter(x_cm.reshape(-1), edge_index.reshape(-1))
    hg = _gate_mlp(aggp, x_cm, Wg1, bg1)
    gp, cnt = _segsum_cnt(hg, batch_pad)
    wts = _gate_head(gp, cnt, Wg2, bg2)
    wnode = _wgather(wts.reshape(-1), batch_pad).reshape(N_PAD, NE)
    y = _expert_mlp(x_cm, wnode, W1, b1, W2, b2, W3, b3)
    (out,) = _segsum_plain(y, batch_pad)
    return out
